# baseline (speedup 1.0000x reference)
import functools

import jax
import jax.numpy as jnp
from jax.experimental import pallas as pl
from jax.experimental.pallas import tpu as pltpu

T = 4096
D = 768
DFF = 3072
E = 8
TOPK = 2

NC = 2
TC_ = T // NC
DFFB = 512
NF = DFF // DFFB


def _router_block(x_ref, wg_ref, bg_ref, wn_ref, bn_ref, noise_ref, g_ref):
    x = x_ref[...]
    logits = jnp.dot(x, wg_ref[...], preferred_element_type=jnp.float32) + bg_ref[0]
    nlog = jnp.dot(x, wn_ref[...], preferred_element_type=jnp.float32) + bn_ref[0]
    noisy = logits + noise_ref[...] * jax.nn.softplus(nlog)

    lane = jax.lax.broadcasted_iota(jnp.int32, noisy.shape, 1)
    m1 = jnp.max(noisy, axis=1, keepdims=True)
    i1 = jnp.min(jnp.where(noisy == m1, lane, E), axis=1, keepdims=True)
    rest = jnp.where(lane == i1, -jnp.inf, noisy)
    m2 = jnp.max(rest, axis=1, keepdims=True)
    i2 = jnp.min(jnp.where(rest == m2, lane, E), axis=1, keepdims=True)
    g1 = 1.0 / (1.0 + jnp.exp(m2 - m1))
    g2 = 1.0 - g1
    g_ref[...] = jnp.where(lane == i1, g1, 0.0) + jnp.where(lane == i2, g2, 0.0)


def _ffn_block(x_ref, g_ref, w1_ref, b1_ref, w2_ref, b2_ref, out_ref):
    e = pl.program_id(1)
    f = pl.program_id(2)

    @pl.when((e == 0) & (f == 0))
    def _():
        out_ref[...] = jnp.zeros_like(out_ref)

    x = x_ref[...].astype(jnp.bfloat16)
    w1 = w1_ref[0].astype(jnp.bfloat16)
    h = jnp.dot(x, w1, preferred_element_type=jnp.float32) + b1_ref[0]
    h = jnp.maximum(h, 0.0).astype(jnp.bfloat16)
    w2 = w2_ref[0].astype(jnp.bfloat16)
    y = jnp.dot(h, w2, preferred_element_type=jnp.float32)

    lane = jax.lax.broadcasted_iota(jnp.int32, g_ref.shape, 1)
    g = jnp.sum(jnp.where(lane == e, g_ref[...], 0.0), axis=1, keepdims=True)

    @pl.when(f == 0)
    def _():
        y_plus = y + b2_ref[0]
        out_ref[...] += g * y_plus

    @pl.when(f != 0)
    def _():
        out_ref[...] += g * y


def kernel(x, Wg, bg, Wn, bn, W1, b1, W2, b2):
    base_noise = jax.random.normal(jax.random.key(42), (T, E), dtype=jnp.float32)

    gating = pl.pallas_call(
        _router_block,
        grid=(T // 512,),
        in_specs=[
            pl.BlockSpec((512, D), lambda t: (t, 0)),
            pl.BlockSpec((D, E), lambda t: (0, 0)),
            pl.BlockSpec((1, E), lambda t: (0, 0)),
            pl.BlockSpec((D, E), lambda t: (0, 0)),
            pl.BlockSpec((1, E), lambda t: (0, 0)),
            pl.BlockSpec((512, E), lambda t: (t, 0)),
        ],
        out_specs=pl.BlockSpec((512, E), lambda t: (t, 0)),
        out_shape=jax.ShapeDtypeStruct((T, E), jnp.float32),
    )(x, Wg, bg[None, :], Wn, bn[None, :], base_noise)

    out = pl.pallas_call(
        _ffn_block,
        grid=(NC, E, NF),
        in_specs=[
            pl.BlockSpec((TC_, D), lambda c, e, f: (c, 0)),
            pl.BlockSpec((TC_, E), lambda c, e, f: (c, 0)),
            pl.BlockSpec((1, D, DFFB), lambda c, e, f: (e, 0, f)),
            pl.BlockSpec((1, 1, DFFB), lambda c, e, f: (e, 0, f)),
            pl.BlockSpec((1, DFFB, D), lambda c, e, f: (e, f, 0)),
            pl.BlockSpec((1, 1, D), lambda c, e, f: (e, 0, 0)),
        ],
        out_specs=pl.BlockSpec((TC_, D), lambda c, e, f: (c, 0)),
        out_shape=jax.ShapeDtypeStruct((T, D), jnp.float32),
        compiler_params=pltpu.CompilerParams(
            dimension_semantics=("parallel", "arbitrary", "arbitrary"),
        ),
    )(x, gating, W1, b1[:, None, :], W2, b2[:, None, :])
    return out
